# baseline (speedup 1.0000x reference)
import functools

import jax
import jax.numpy as jnp
from jax import lax
from jax.experimental import pallas as pl
from jax.experimental.pallas import tpu as pltpu
from jax.experimental.pallas import tpu_sc as plsc

N = 10000
D = 128
E = 320000
K = 3

NC = 2
NS = 16
L = 16
NW = NC * NS

NPAD = 10240
C = 128
CH = 80
EPW = C * CH
EPAD = NW * EPW
TOTCH = EPAD // C
CH0 = 152
CH1 = 8
BS = 8
TOTCH_ALLOC = TOTCH + (CH0 - CH1)
RPW = NPAD // NW
RPT = NPAD // NS

_mesh = plsc.VectorSubcoreMesh(core_axis_name="c", subcore_axis_name="s")


def _zero_vec():
    return jnp.zeros((L,), jnp.float32)


@functools.partial(
    pl.kernel,
    out_type=jax.ShapeDtypeStruct((NC * NPAD,), jnp.float32),
    mesh=_mesh,
    scratch_types=[
        pltpu.VMEM((CH, C), jnp.int32),
        pltpu.VMEM((CH, C), jnp.float32),
        pltpu.VMEM((RPT,), jnp.float32),
        pltpu.VMEM_SHARED((NPAD,), jnp.float32),
    ],
)
def _sc_deg(col_hbm, ew_hbm, deg_hbm, col_v, ew_v, z_v, deg_sh):
    cid = lax.axis_index("c")
    sid = lax.axis_index("s")
    w = sid * NC + cid
    pltpu.sync_copy(col_hbm.at[pl.ds(w * CH, CH)], col_v)
    pltpu.sync_copy(ew_hbm.at[pl.ds(w * CH, CH)], ew_v)

    def zr(i, carry):
        z_v[pl.ds(i * L, L)] = _zero_vec()
        return carry

    lax.fori_loop(0, RPT // L, zr, 0, unroll=4)
    pltpu.sync_copy(z_v, deg_sh.at[pl.ds(sid * RPT, RPT)])
    plsc.subcore_barrier()

    def chunk(j, carry):
        pltpu.sync_copy(ew_v.at[j], deg_sh.at[col_v.at[j]], add=True)
        return carry

    lax.fori_loop(0, CH, chunk, 0)
    plsc.subcore_barrier()
    pltpu.sync_copy(deg_sh.at[pl.ds(sid * RPT, RPT)],
                    deg_hbm.at[pl.ds(cid * NPAD + sid * RPT, RPT)])


@functools.partial(
    pl.kernel,
    out_type=jax.ShapeDtypeStruct((NC, NPAD, D), jnp.float32),
    mesh=_mesh,
    scratch_types=[
        pltpu.VMEM((2, BS, C), jnp.int32),
        pltpu.VMEM((2, BS, C), jnp.int32),
        pltpu.VMEM((2, BS, C), jnp.float32),
        pltpu.VMEM((2, C, D), jnp.float32),
        pltpu.VMEM_SHARED((NPAD, D), jnp.float32),
        pltpu.SemaphoreType.DMA,
        pltpu.SemaphoreType.DMA,
        pltpu.SemaphoreType.DMA,
        pltpu.SemaphoreType.DMA,
        pltpu.SemaphoreType.DMA,
    ],
)
def _sc_hop(g_hbm, row_hbm, col_hbm, ew_hbm, p_hbm,
            rowb, colb, ewb, rows_v, acc_sh, gsem0, gsem1, ssem0, ssem1, stsem):
    cid = lax.axis_index("c")
    sid = lax.axis_index("s")
    nch = jnp.where(cid == 0, CH0, CH1)
    base = jnp.where(cid == 0, sid * CH0, 16 * CH0 + sid * CH1)

    def zrow(i, carry):
        for k in range(D // L):
            rows_v[0, i, pl.ds(k * L, L)] = _zero_vec()
        return carry

    lax.fori_loop(0, C, zrow, 0, unroll=2)
    for bblk in range(RPT // C):
        pltpu.sync_copy(rows_v.at[0], acc_sh.at[pl.ds(sid * RPT + bblk * C, C)])
    plsc.subcore_barrier()

    def stage_block(b, sync):
        bslot = lax.rem(b, 2)
        src = pl.ds(base + b * BS, BS)
        if sync:
            pltpu.sync_copy(row_hbm.at[src], rowb.at[bslot])
            pltpu.sync_copy(col_hbm.at[src], colb.at[bslot])
            pltpu.sync_copy(ew_hbm.at[src], ewb.at[bslot])
        else:
            pltpu.async_copy(row_hbm.at[src], rowb.at[bslot], stsem)
            pltpu.async_copy(col_hbm.at[src], colb.at[bslot], stsem)
            pltpu.async_copy(ew_hbm.at[src], ewb.at[bslot], stsem)

    def wait_block(b):
        bslot = lax.rem(b, 2)
        src = pl.ds(base + b * BS, BS)
        pltpu.make_async_copy(row_hbm.at[src], rowb.at[bslot], stsem).wait()
        pltpu.make_async_copy(col_hbm.at[src], colb.at[bslot], stsem).wait()
        pltpu.make_async_copy(ew_hbm.at[src], ewb.at[bslot], stsem).wait()

    def gather(j, slot):
        bslot = lax.rem(j // BS, 2)
        pltpu.async_copy(g_hbm.at[rowb.at[bslot, lax.rem(j, BS)]],
                         rows_v.at[slot], gsem0 if slot == 0 else gsem1)

    def wait_gather(j, slot):
        bslot = lax.rem(j // BS, 2)
        pltpu.make_async_copy(g_hbm.at[rowb.at[bslot, lax.rem(j, BS)]],
                              rows_v.at[slot],
                              gsem0 if slot == 0 else gsem1).wait()

    def scale(j, slot):
        bslot = lax.rem(j // BS, 2)
        jm = lax.rem(j, BS)

        def egroup(eb, ecarry):
            ew16 = ewb[bslot, jm, pl.ds(eb * L, L)]
            for lane in range(L):
                e = eb * L + lane
                s = jnp.broadcast_to(ew16[lane], (L,))
                for k in range(D // L):
                    rows_v[slot, e, pl.ds(k * L, L)] = (
                        rows_v[slot, e, pl.ds(k * L, L)] * s
                    )
            return ecarry

        lax.fori_loop(0, C // L, egroup, 0, unroll=2)

    def scatter(j, slot):
        bslot = lax.rem(j // BS, 2)
        pltpu.async_copy(rows_v.at[slot],
                         acc_sh.at[colb.at[bslot, lax.rem(j, BS)]],
                         ssem0 if slot == 0 else ssem1, add=True)

    def wait_scatter(j, slot):
        bslot = lax.rem(j // BS, 2)
        pltpu.make_async_copy(rows_v.at[slot],
                              acc_sh.at[colb.at[bslot, lax.rem(j, BS)]],
                              ssem0 if slot == 0 else ssem1).wait()

    stage_block(0, True)
    gather(0, 0)

    def pair(jj, carry):
        j0 = 2 * jj
        j1 = j0 + 1
        wait_gather(j0, 0)

        @pl.when(jj > 0)
        def _():
            wait_scatter(j1 - 2, 1)

        @pl.when((lax.rem(j0, BS) == 0) & ((j0 // BS + 1) * BS < nch))
        def _():
            stage_block(j0 // BS + 1, False)

        gather(j1, 1)
        scale(j0, 0)
        scatter(j0, 0)
        wait_gather(j1, 1)

        @pl.when(j1 + 1 < nch)
        def _():
            wait_scatter(j0, 0)

            @pl.when(lax.rem(j1 + 1, BS) == 0)
            def _():
                wait_block((j1 + 1) // BS)

            gather(j1 + 1, 0)

        scale(j1, 1)
        scatter(j1, 1)
        return carry

    lax.fori_loop(0, nch // 2, pair, 0)
    wait_scatter(nch - 2, 0)
    wait_scatter(nch - 1, 1)
    plsc.subcore_barrier()
    pltpu.sync_copy(acc_sh.at[pl.ds(sid * RPT, RPT)],
                    p_hbm.at[cid, pl.ds(sid * RPT, RPT)])


R = 256
GRID = NPAD // R


def _dis_col(deg_ref):
    dsum = deg_ref[:, 0:1] + deg_ref[:, 1:2]
    return jnp.where(dsum > 0.0, lax.rsqrt(jnp.where(dsum > 0.0, dsum, 1.0)), 0.0)


def _tc_init_body(x_ref, dis_ref, w_ref, b_ref, out_ref, g_ref):
    xb = x_ref[...]
    d = _dis_col(dis_ref)
    g_ref[...] = xb * d
    out_ref[...] = (
        jnp.dot(xb, w_ref[...], preferred_element_type=jnp.float32) + b_ref[...]
    )


_tc_init = pl.pallas_call(
    _tc_init_body,
    grid=(GRID,),
    in_specs=[
        pl.BlockSpec((R, D), lambda i: (i, 0)),
        pl.BlockSpec((R, NC), lambda i: (i, 0)),
        pl.BlockSpec((D, D), lambda i: (0, 0)),
        pl.BlockSpec((1, D), lambda i: (0, 0)),
    ],
    out_specs=[pl.BlockSpec((R, D), lambda i: (i, 0))] * 2,
    out_shape=[jax.ShapeDtypeStruct((NPAD, D), jnp.float32)] * 2,
)


def _tc_hop_body(p_ref, dis_ref, out_in_ref, w_ref, b_ref, out_ref, g_ref):
    p = p_ref[0] + p_ref[1]
    d = _dis_col(dis_ref)
    h = p * d
    g_ref[...] = h * d
    out_ref[...] = (
        out_in_ref[...]
        + jnp.dot(h, w_ref[...], preferred_element_type=jnp.float32)
        + b_ref[...]
    )


_tc_hop = pl.pallas_call(
    _tc_hop_body,
    grid=(GRID,),
    in_specs=[
        pl.BlockSpec((NC, R, D), lambda i: (0, i, 0)),
        pl.BlockSpec((R, NC), lambda i: (i, 0)),
        pl.BlockSpec((R, D), lambda i: (i, 0)),
        pl.BlockSpec((D, D), lambda i: (0, 0)),
        pl.BlockSpec((1, D), lambda i: (0, 0)),
    ],
    out_specs=[pl.BlockSpec((R, D), lambda i: (i, 0))] * 2,
    out_shape=[jax.ShapeDtypeStruct((NPAD, D), jnp.float32)] * 2,
)


def kernel(x, edge_index, edge_attr, W, b):
    row = edge_index[0]
    col = edge_index[1]
    pad_e = TOTCH_ALLOC * C - E
    row2 = jnp.pad(row, (0, pad_e)).reshape(TOTCH_ALLOC, C)
    col2 = jnp.pad(col, (0, pad_e)).reshape(TOTCH_ALLOC, C)
    ew2 = jnp.pad(edge_attr, (0, pad_e)).reshape(TOTCH_ALLOC, C)
    xp = jnp.pad(x, ((0, NPAD - N), (0, 0)))
    b2 = b.reshape(K + 1, 1, D)

    deg2 = _sc_deg(col2, ew2)
    dis1 = deg2.reshape(NC, NPAD).T

    out, g = _tc_init(xp, dis1, W[0], b2[0])
    for k in range(1, K + 1):
        p2 = _sc_hop(g, row2, col2, ew2)
        out, g = _tc_hop(p2, dis1, out, W[k], b2[k])
    return out[:N]


# speedup vs baseline: 1.0166x; 1.0166x over previous
import functools

import jax
import jax.numpy as jnp
from jax import lax
from jax.experimental import pallas as pl
from jax.experimental.pallas import tpu as pltpu
from jax.experimental.pallas import tpu_sc as plsc

N = 10000
D = 128
E = 320000
K = 3

NC = 2
NS = 16
L = 16
NW = NC * NS

NPAD = 10240
C = 128
CH = 80
EPW = C * CH
EPAD = NW * EPW
TOTCH = EPAD // C
CH0 = 144
CH1 = 16
BS = 8
TOTCH_ALLOC = TOTCH + (CH0 - CH1)
RPW = NPAD // NW
RPT = NPAD // NS

_mesh = plsc.VectorSubcoreMesh(core_axis_name="c", subcore_axis_name="s")


def _zero_vec():
    return jnp.zeros((L,), jnp.float32)


@functools.partial(
    pl.kernel,
    out_type=jax.ShapeDtypeStruct((NC * NPAD,), jnp.float32),
    mesh=_mesh,
    scratch_types=[
        pltpu.VMEM((CH, C), jnp.int32),
        pltpu.VMEM((CH, C), jnp.float32),
        pltpu.VMEM((RPT,), jnp.float32),
        pltpu.VMEM_SHARED((NPAD,), jnp.float32),
    ],
)
def _sc_deg(col_hbm, ew_hbm, deg_hbm, col_v, ew_v, z_v, deg_sh):
    cid = lax.axis_index("c")
    sid = lax.axis_index("s")
    w = sid * NC + cid
    pltpu.sync_copy(col_hbm.at[pl.ds(w * CH, CH)], col_v)
    pltpu.sync_copy(ew_hbm.at[pl.ds(w * CH, CH)], ew_v)

    def zr(i, carry):
        z_v[pl.ds(i * L, L)] = _zero_vec()
        return carry

    lax.fori_loop(0, RPT // L, zr, 0, unroll=4)
    pltpu.sync_copy(z_v, deg_sh.at[pl.ds(sid * RPT, RPT)])
    plsc.subcore_barrier()

    def chunk(j, carry):
        pltpu.sync_copy(ew_v.at[j], deg_sh.at[col_v.at[j]], add=True)
        return carry

    lax.fori_loop(0, CH, chunk, 0)
    plsc.subcore_barrier()
    pltpu.sync_copy(deg_sh.at[pl.ds(sid * RPT, RPT)],
                    deg_hbm.at[pl.ds(cid * NPAD + sid * RPT, RPT)])


@functools.partial(
    pl.kernel,
    out_type=jax.ShapeDtypeStruct((NC, NPAD, D), jnp.float32),
    mesh=_mesh,
    scratch_types=[
        pltpu.VMEM((2, BS, C), jnp.int32),
        pltpu.VMEM((2, BS, C), jnp.int32),
        pltpu.VMEM((2, BS, C), jnp.float32),
        pltpu.VMEM((2, C, D), jnp.float32),
        pltpu.VMEM_SHARED((NPAD, D), jnp.float32),
        pltpu.SemaphoreType.DMA,
        pltpu.SemaphoreType.DMA,
        pltpu.SemaphoreType.DMA,
        pltpu.SemaphoreType.DMA,
        pltpu.SemaphoreType.DMA,
    ],
)
def _sc_hop(g_hbm, row_hbm, col_hbm, ew_hbm, p_hbm,
            rowb, colb, ewb, rows_v, acc_sh, gsem0, gsem1, ssem0, ssem1, stsem):
    cid = lax.axis_index("c")
    sid = lax.axis_index("s")
    nch = jnp.where(cid == 0, CH0, CH1)
    base = jnp.where(cid == 0, sid * CH0, 16 * CH0 + sid * CH1)

    def zrow(i, carry):
        for k in range(D // L):
            rows_v[0, i, pl.ds(k * L, L)] = _zero_vec()
        return carry

    lax.fori_loop(0, C, zrow, 0, unroll=2)
    for bblk in range(RPT // C):
        pltpu.sync_copy(rows_v.at[0], acc_sh.at[pl.ds(sid * RPT + bblk * C, C)])
    plsc.subcore_barrier()

    def stage_block(b, sync):
        bslot = lax.rem(b, 2)
        src = pl.ds(base + b * BS, BS)
        if sync:
            pltpu.sync_copy(row_hbm.at[src], rowb.at[bslot])
            pltpu.sync_copy(col_hbm.at[src], colb.at[bslot])
            pltpu.sync_copy(ew_hbm.at[src], ewb.at[bslot])
        else:
            pltpu.async_copy(row_hbm.at[src], rowb.at[bslot], stsem)
            pltpu.async_copy(col_hbm.at[src], colb.at[bslot], stsem)
            pltpu.async_copy(ew_hbm.at[src], ewb.at[bslot], stsem)

    def wait_block(b):
        bslot = lax.rem(b, 2)
        src = pl.ds(base + b * BS, BS)
        pltpu.make_async_copy(row_hbm.at[src], rowb.at[bslot], stsem).wait()
        pltpu.make_async_copy(col_hbm.at[src], colb.at[bslot], stsem).wait()
        pltpu.make_async_copy(ew_hbm.at[src], ewb.at[bslot], stsem).wait()

    def gather(j, slot):
        bslot = lax.rem(j // BS, 2)
        pltpu.async_copy(g_hbm.at[rowb.at[bslot, lax.rem(j, BS)]],
                         rows_v.at[slot], gsem0 if slot == 0 else gsem1)

    def wait_gather(j, slot):
        bslot = lax.rem(j // BS, 2)
        pltpu.make_async_copy(g_hbm.at[rowb.at[bslot, lax.rem(j, BS)]],
                              rows_v.at[slot],
                              gsem0 if slot == 0 else gsem1).wait()

    def scale(j, slot):
        bslot = lax.rem(j // BS, 2)
        jm = lax.rem(j, BS)

        def egroup(eb, ecarry):
            ew16 = ewb[bslot, jm, pl.ds(eb * L, L)]
            for lane in range(L):
                e = eb * L + lane
                s = jnp.broadcast_to(ew16[lane], (L,))
                for k in range(D // L):
                    rows_v[slot, e, pl.ds(k * L, L)] = (
                        rows_v[slot, e, pl.ds(k * L, L)] * s
                    )
            return ecarry

        lax.fori_loop(0, C // L, egroup, 0, unroll=2)

    def scatter(j, slot):
        bslot = lax.rem(j // BS, 2)
        pltpu.async_copy(rows_v.at[slot],
                         acc_sh.at[colb.at[bslot, lax.rem(j, BS)]],
                         ssem0 if slot == 0 else ssem1, add=True)

    def wait_scatter(j, slot):
        bslot = lax.rem(j // BS, 2)
        pltpu.make_async_copy(rows_v.at[slot],
                              acc_sh.at[colb.at[bslot, lax.rem(j, BS)]],
                              ssem0 if slot == 0 else ssem1).wait()

    stage_block(0, True)
    gather(0, 0)

    def pair(jj, carry):
        j0 = 2 * jj
        j1 = j0 + 1
        wait_gather(j0, 0)

        @pl.when(jj > 0)
        def _():
            wait_scatter(j1 - 2, 1)

        @pl.when((lax.rem(j0, BS) == 0) & ((j0 // BS + 1) * BS < nch))
        def _():
            stage_block(j0 // BS + 1, False)

        gather(j1, 1)
        scale(j0, 0)
        scatter(j0, 0)
        wait_gather(j1, 1)

        @pl.when(j1 + 1 < nch)
        def _():
            wait_scatter(j0, 0)

            @pl.when(lax.rem(j1 + 1, BS) == 0)
            def _():
                wait_block((j1 + 1) // BS)

            gather(j1 + 1, 0)

        scale(j1, 1)
        scatter(j1, 1)
        return carry

    lax.fori_loop(0, nch // 2, pair, 0)
    wait_scatter(nch - 2, 0)
    wait_scatter(nch - 1, 1)
    plsc.subcore_barrier()
    pltpu.sync_copy(acc_sh.at[pl.ds(sid * RPT, RPT)],
                    p_hbm.at[cid, pl.ds(sid * RPT, RPT)])


R = 256
GRID = NPAD // R


def _dis_col(deg_ref):
    dsum = deg_ref[:, 0:1] + deg_ref[:, 1:2]
    return jnp.where(dsum > 0.0, lax.rsqrt(jnp.where(dsum > 0.0, dsum, 1.0)), 0.0)


_ROW_SPEC = pl.BlockSpec((R, D), lambda i: (i, 0))
_DIS_SPEC = pl.BlockSpec((R, NC), lambda i: (i, 0))
_P_SPEC = pl.BlockSpec((NC, R, D), lambda i: (0, i, 0))
_W_SPEC = pl.BlockSpec((D, D), lambda i: (0, 0))
_B_SPEC = pl.BlockSpec((1, D), lambda i: (0, 0))
_ROW_OUT = jax.ShapeDtypeStruct((NPAD, D), jnp.float32)


def _tc_g0_body(x_ref, dis_ref, g_ref):
    g_ref[...] = x_ref[...] * _dis_col(dis_ref)


_tc_g0 = pl.pallas_call(
    _tc_g0_body,
    grid=(GRID,),
    in_specs=[_ROW_SPEC, _DIS_SPEC],
    out_specs=_ROW_SPEC,
    out_shape=_ROW_OUT,
)


def _tc_g_body(p_ref, dis_ref, g_ref):
    d = _dis_col(dis_ref)
    g_ref[...] = (p_ref[0] + p_ref[1]) * d * d


_tc_g = pl.pallas_call(
    _tc_g_body,
    grid=(GRID,),
    in_specs=[_P_SPEC, _DIS_SPEC],
    out_specs=_ROW_SPEC,
    out_shape=_ROW_OUT,
)


def _tc_out0_body(x_ref, w_ref, b_ref, out_ref):
    out_ref[...] = (
        jnp.dot(x_ref[...], w_ref[...], preferred_element_type=jnp.float32)
        + b_ref[...]
    )


_tc_out0 = pl.pallas_call(
    _tc_out0_body,
    grid=(GRID,),
    in_specs=[_ROW_SPEC, _W_SPEC, _B_SPEC],
    out_specs=_ROW_SPEC,
    out_shape=_ROW_OUT,
)


def _tc_out_body(p_ref, dis_ref, out_in_ref, w_ref, b_ref, out_ref):
    h = (p_ref[0] + p_ref[1]) * _dis_col(dis_ref)
    out_ref[...] = (
        out_in_ref[...]
        + jnp.dot(h, w_ref[...], preferred_element_type=jnp.float32)
        + b_ref[...]
    )


_tc_out = pl.pallas_call(
    _tc_out_body,
    grid=(GRID,),
    in_specs=[_P_SPEC, _DIS_SPEC, _ROW_SPEC, _W_SPEC, _B_SPEC],
    out_specs=_ROW_SPEC,
    out_shape=_ROW_OUT,
)


def kernel(x, edge_index, edge_attr, W, b):
    row = edge_index[0]
    col = edge_index[1]
    pad_e = TOTCH_ALLOC * C - E
    row2 = jnp.pad(row, (0, pad_e)).reshape(TOTCH_ALLOC, C)
    col2 = jnp.pad(col, (0, pad_e)).reshape(TOTCH_ALLOC, C)
    ew2 = jnp.pad(edge_attr, (0, pad_e)).reshape(TOTCH_ALLOC, C)
    xp = jnp.pad(x, ((0, NPAD - N), (0, 0)))
    b2 = b.reshape(K + 1, 1, D)

    deg2 = _sc_deg(col2, ew2)
    dis1 = deg2.reshape(NC, NPAD).T

    g = _tc_g0(xp, dis1)
    out = _tc_out0(xp, W[0], b2[0])
    for k in range(1, K + 1):
        p2 = _sc_hop(g, row2, col2, ew2)
        if k < K:
            g = _tc_g(p2, dis1)
        out = _tc_out(p2, dis1, out, W[k], b2[k])
    return out[:N]
